# baseline (speedup 1.0000x reference)
import functools

import jax
import jax.numpy as jnp
from jax import lax
from jax.experimental import pallas as pl
from jax.experimental.pallas import tpu as pltpu
from jax.experimental.pallas import tpu_sc as plsc


_NC = 2
_NS = 16
_NW = _NC * _NS
_CH = 200
_NBUF = 4
_D = _NBUF // 2


def kernel(x):
    n, d = x.shape
    nch = n // _CH
    mesh = plsc.VectorSubcoreMesh(core_axis_name="c", subcore_axis_name="s")

    @functools.partial(
        pl.kernel,
        out_type=jax.ShapeDtypeStruct((n, d), x.dtype),
        mesh=mesh,
        scratch_types=[
            pltpu.VMEM((_NBUF, _CH, d), x.dtype),
            pltpu.SemaphoreType.DMA((_NBUF,)),
            pltpu.SemaphoreType.DMA((_NBUF,)),
        ],
    )
    def _copy(x_hbm, o_hbm, bufs, in_sems, out_sems):
        wid = lax.axis_index("s") * _NC + lax.axis_index("c")
        n_my = (nch - wid + _NW - 1) // _NW

        def in_cp(k):
            b = lax.rem(k, _NBUF)
            row = (wid + k * _NW) * _CH
            return pltpu.make_async_copy(
                x_hbm.at[pl.ds(row, _CH), :], bufs.at[b], in_sems.at[b]
            )

        def out_cp(k):
            b = lax.rem(k, _NBUF)
            row = (wid + k * _NW) * _CH
            return pltpu.make_async_copy(
                bufs.at[b], o_hbm.at[pl.ds(row, _CH), :], out_sems.at[b]
            )

        for k in range(_D):
            @pl.when(k < n_my)
            def _():
                in_cp(k).start()

        def step(k, carry):
            @pl.when(k - _D >= 0)
            def _():
                out_cp(k - _D).wait()

            @pl.when(k + _D < n_my)
            def _():
                in_cp(k + _D).start()

            in_cp(k).wait()

            @pl.when(jnp.logical_and(wid == 0, k == 0))
            def _():
                ones = jnp.ones((16,), x.dtype)
                for r in range(2):
                    for j in range(d // 16):
                        bufs[0, r, pl.ds(16 * j, 16)] = ones

            out_cp(k).start()
            return carry

        lax.fori_loop(0, n_my, step, 0)

        for j in range(_D):
            @pl.when(n_my - _D + j >= 0)
            def _():
                out_cp(n_my - _D + j).wait()

    return _copy(x)


# speedup vs baseline: 1.5481x; 1.5481x over previous
import jax
import jax.numpy as jnp
from jax.experimental import pallas as pl
from jax.experimental.pallas import tpu as pltpu


def _body(x_ref, o_ref, ones_vmem, sem):
    del x_ref
    ones_vmem[...] = jnp.ones_like(ones_vmem)
    cp = pltpu.make_async_copy(
        ones_vmem, o_ref.at[pl.ds(0, ones_vmem.shape[0]), :], sem
    )
    cp.start()
    cp.wait()


def kernel(x):
    n, d = x.shape
    return pl.pallas_call(
        _body,
        in_specs=[pl.BlockSpec(memory_space=pltpu.MemorySpace.HBM)],
        out_specs=pl.BlockSpec(memory_space=pltpu.MemorySpace.HBM),
        out_shape=jax.ShapeDtypeStruct((n, d), x.dtype),
        input_output_aliases={0: 0},
        scratch_shapes=[
            pltpu.VMEM((2, d), x.dtype),
            pltpu.SemaphoreType.DMA,
        ],
    )(x)
